# speedup vs baseline: 2.1712x; 2.1712x over previous
import jax
import jax.numpy as jnp
from jax.experimental import pallas as pl

D_MODEL = 768
SEQ_BLOCK = 512


def _body(x_ref, abs_ref, m0_ref, m1_ref, m2_ref, out_ref):
    sb = pl.program_id(0)
    rows = x_ref.shape[1]
    pos = sb * rows + jax.lax.broadcasted_iota(jnp.int32, (rows, 1), 0)

    abs_blk = abs_ref[...]

    r2 = pos % 2
    pe = abs_blk + jnp.where(r2 == 0, m0_ref[0:1, :], m0_ref[1:2, :])

    r3 = pos % 3
    pe = pe + jnp.where(
        r3 == 0, m1_ref[0:1, :], jnp.where(r3 == 1, m1_ref[1:2, :], m1_ref[2:3, :])
    )

    r4 = pos % 4
    pe = pe + jnp.where(
        r4 == 0,
        m2_ref[0:1, :],
        jnp.where(r4 == 1, m2_ref[1:2, :], jnp.where(r4 == 2, m2_ref[2:3, :], m2_ref[3:4, :])),
    )

    out_ref[...] = x_ref[...] + pe[None, :, :]


def kernel(x, abs_table, mod_table_0, mod_table_1, mod_table_2):
    batch, seq_len, d = x.shape
    n_sb = seq_len // SEQ_BLOCK
    grid = (n_sb, batch)
    return pl.pallas_call(
        _body,
        grid=grid,
        in_specs=[
            pl.BlockSpec((1, SEQ_BLOCK, d), lambda sb, b: (b, sb, 0)),
            pl.BlockSpec((SEQ_BLOCK, d), lambda sb, b: (sb, 0)),
            pl.BlockSpec((2, d), lambda sb, b: (0, 0)),
            pl.BlockSpec((3, d), lambda sb, b: (0, 0)),
            pl.BlockSpec((4, d), lambda sb, b: (0, 0)),
        ],
        out_specs=pl.BlockSpec((1, SEQ_BLOCK, d), lambda sb, b: (b, sb, 0)),
        out_shape=jax.ShapeDtypeStruct(x.shape, x.dtype),
    )(x, abs_table, mod_table_0, mod_table_1, mod_table_2)
